# Initial kernel scaffold; baseline (speedup 1.0000x reference)
import jax
import jax.numpy as jnp
from jax.experimental import pallas as pl


def kernel(X, E):
    raise NotImplementedError("write your pallas kernel here")


# speedup vs baseline: 1.3067x; 1.3067x over previous
import functools

import jax
import jax.numpy as jnp
from jax import lax
from jax.experimental import pallas as pl
from jax.experimental.pallas import tpu as pltpu
from jax.experimental.pallas import tpu_sc as plsc

_NW = 32
_CH = 128


def _emb_call(B, D, V):
    b_per_w = B // _NW
    n_ch = b_per_w // _CH
    mesh = plsc.VectorSubcoreMesh(core_axis_name="c", subcore_axis_name="s")

    @functools.partial(
        pl.kernel,
        mesh=mesh,
        out_type=jax.ShapeDtypeStruct((B, D), jnp.float32),
        scratch_types=[
            pltpu.VMEM((n_ch, _CH), jnp.int32),
            pltpu.VMEM((_CH, D), jnp.float32),
            pltpu.SemaphoreType.DMA,
        ],
        compiler_params=pltpu.CompilerParams(use_tc_tiling_on_sc=False),
    )
    def emb(table_hbm, idx_hbm, out_hbm, idx_v, rows_v, gsem):
        wid = lax.axis_index("s") * 2 + lax.axis_index("c")
        base = wid * b_per_w
        pltpu.sync_copy(idx_hbm.at[wid], idx_v)

        def body(j, carry):
            pltpu.async_copy(table_hbm.at[idx_v.at[j]], rows_v, gsem).wait()
            pltpu.sync_copy(rows_v, out_hbm.at[pl.ds(base + j * _CH, _CH)])
            return carry

        lax.fori_loop(0, n_ch, body, 0)

    return emb


def kernel(X, E):
    B = X.shape[0] * X.shape[1]
    D = E.shape[1]
    x3 = X.reshape(_NW, (B // _NW) // _CH, _CH)
    out = _emb_call(B, D, E.shape[0])(E, x3)
    return out.reshape(X.shape[0], X.shape[1], D)


# speedup vs baseline: 1.4972x; 1.1458x over previous
import functools

import jax
import jax.numpy as jnp
from jax import lax
from jax.experimental import pallas as pl
from jax.experimental.pallas import tpu as pltpu
from jax.experimental.pallas import tpu_sc as plsc

_NW = 32
_CH = 128
_NB = 8
_LA = 4


def _emb_call(B, D):
    b_per_w = B // _NW
    n_ch = b_per_w // _CH
    assert n_ch % _NB == 0 and n_ch >= 2 * _NB
    mesh = plsc.VectorSubcoreMesh(core_axis_name="c", subcore_axis_name="s")

    @functools.partial(
        pl.kernel,
        mesh=mesh,
        out_type=jax.ShapeDtypeStruct((B, D), jnp.float32),
        scratch_types=[
            pltpu.VMEM((n_ch, _CH), jnp.int32),
            pltpu.VMEM((_NB, _CH, D), jnp.float32),
            [pltpu.SemaphoreType.DMA] * _NB,
            [pltpu.SemaphoreType.DMA] * _NB,
        ],
        compiler_params=pltpu.CompilerParams(use_tc_tiling_on_sc=False),
    )
    def emb(table_hbm, idx_hbm, out_hbm, idx_v, rows_v, gsems, osems):
        wid = lax.axis_index("s") * 2 + lax.axis_index("c")
        base = wid * b_per_w
        pltpu.sync_copy(idx_hbm.at[wid], idx_v)

        def g_fire(j, b):
            pltpu.async_copy(table_hbm.at[idx_v.at[j]], rows_v.at[b], gsems[b])

        def g_wait(j, b):
            pltpu.make_async_copy(
                table_hbm.at[idx_v.at[j]], rows_v.at[b], gsems[b]).wait()

        def o_fire(j, b):
            pltpu.async_copy(
                rows_v.at[b], out_hbm.at[pl.ds(base + j * _CH, _CH)], osems[b])

        def o_wait(j, b):
            pltpu.make_async_copy(
                rows_v.at[b], out_hbm.at[pl.ds(base + j * _CH, _CH)],
                osems[b]).wait()

        for b in range(_LA):
            g_fire(b, b)

        def step(j, b, first, last):
            b2 = (b + _LA) % _NB
            g_wait(j, b)
            o_fire(j, b)
            if not first:
                o_wait(j - _LA, b2)
            if not last:
                g_fire(j + _LA, b2)

        for b in range(_NB):
            step(b, b, first=(b < _LA), last=False)

        def group(g, carry):
            j0 = g * _NB
            for b in range(_NB):
                step(j0 + b, b, first=False, last=False)
            return carry

        lax.fori_loop(1, n_ch // _NB - 1, group, 0)

        j0 = n_ch - _NB
        for b in range(_NB):
            step(j0 + b, b, first=False, last=(b >= _NB - _LA))

        for j in range(n_ch - _LA, n_ch):
            o_wait(j, j % _NB)

    return emb


def kernel(X, E):
    B = X.shape[0] * X.shape[1]
    D = E.shape[1]
    x3 = X.reshape(_NW, (B // _NW) // _CH, _CH)
    out = _emb_call(B, D)(E, x3)
    return out.reshape(X.shape[0], X.shape[1], D)


# speedup vs baseline: 1.5010x; 1.0025x over previous
import functools

import jax
import jax.numpy as jnp
from jax import lax
from jax.experimental import pallas as pl
from jax.experimental.pallas import tpu as pltpu
from jax.experimental.pallas import tpu_sc as plsc

_NW = 32
_CH = 256
_NB = 10
_LA = 5


def _emb_call(B, D):
    b_per_w = B // _NW
    n_ch = b_per_w // _CH
    assert n_ch % _NB == 0 and n_ch >= 2 * _NB
    mesh = plsc.VectorSubcoreMesh(core_axis_name="c", subcore_axis_name="s")

    @functools.partial(
        pl.kernel,
        mesh=mesh,
        out_type=jax.ShapeDtypeStruct((B, D), jnp.float32),
        scratch_types=[
            pltpu.VMEM((n_ch, _CH), jnp.int32),
            pltpu.VMEM((_NB, _CH, D), jnp.float32),
            [pltpu.SemaphoreType.DMA] * _NB,
            [pltpu.SemaphoreType.DMA] * _NB,
        ],
        compiler_params=pltpu.CompilerParams(use_tc_tiling_on_sc=False),
    )
    def emb(table_hbm, idx_hbm, out_hbm, idx_v, rows_v, gsems, osems):
        wid = lax.axis_index("s") * 2 + lax.axis_index("c")
        base = wid * b_per_w
        pltpu.sync_copy(idx_hbm.at[wid], idx_v)

        def g_fire(j, b):
            pltpu.async_copy(table_hbm.at[idx_v.at[j]], rows_v.at[b], gsems[b])

        def g_wait(j, b):
            pltpu.make_async_copy(
                table_hbm.at[idx_v.at[j]], rows_v.at[b], gsems[b]).wait()

        def o_fire(j, b):
            pltpu.async_copy(
                rows_v.at[b], out_hbm.at[pl.ds(base + j * _CH, _CH)], osems[b])

        def o_wait(j, b):
            pltpu.make_async_copy(
                rows_v.at[b], out_hbm.at[pl.ds(base + j * _CH, _CH)],
                osems[b]).wait()

        for b in range(_LA):
            g_fire(b, b)

        def step(j, b, first, last):
            b2 = (b + _LA) % _NB
            g_wait(j, b)
            o_fire(j, b)
            if not first:
                o_wait(j - _LA, b2)
            if not last:
                g_fire(j + _LA, b2)

        for b in range(_NB):
            step(b, b, first=(b < _LA), last=False)

        def group(g, carry):
            j0 = g * _NB
            for b in range(_NB):
                step(j0 + b, b, first=False, last=False)
            return carry

        lax.fori_loop(1, n_ch // _NB - 1, group, 0)

        j0 = n_ch - _NB
        for b in range(_NB):
            step(j0 + b, b, first=False, last=(b >= _NB - _LA))

        for j in range(n_ch - _LA, n_ch):
            o_wait(j, j % _NB)

    return emb


def kernel(X, E):
    B = X.shape[0] * X.shape[1]
    D = E.shape[1]
    x3 = X.reshape(_NW, (B // _NW) // _CH, _CH)
    out = _emb_call(B, D)(E, x3)
    return out.reshape(X.shape[0], X.shape[1], D)


# speedup vs baseline: 1.5012x; 1.0001x over previous
import functools

import jax
import jax.numpy as jnp
from jax import lax
from jax.experimental import pallas as pl
from jax.experimental.pallas import tpu as pltpu
from jax.experimental.pallas import tpu_sc as plsc

_NW = 32
_NB = 8
_LA = 4


def _emb_call(M, S, D):
    m_per_w = M // _NW
    assert m_per_w % _NB == 0 and m_per_w >= 2 * _NB
    mesh = plsc.VectorSubcoreMesh(core_axis_name="c", subcore_axis_name="s")

    @functools.partial(
        pl.kernel,
        mesh=mesh,
        out_type=jax.ShapeDtypeStruct((M, S, D), jnp.float32),
        scratch_types=[
            pltpu.VMEM((m_per_w, S), jnp.int32),
            pltpu.VMEM((_NB, S, D), jnp.float32),
            [pltpu.SemaphoreType.DMA] * _NB,
            [pltpu.SemaphoreType.DMA] * _NB,
        ],
        compiler_params=pltpu.CompilerParams(use_tc_tiling_on_sc=False),
    )
    def emb(table_hbm, idx_hbm, out_hbm, idx_v, rows_v, gsems, osems):
        wid = lax.axis_index("s") * 2 + lax.axis_index("c")
        base = wid * m_per_w
        pltpu.sync_copy(idx_hbm.at[pl.ds(base, m_per_w)], idx_v)

        def g_fire(j, b):
            pltpu.async_copy(table_hbm.at[idx_v.at[j]], rows_v.at[b], gsems[b])

        def g_wait(j, b):
            pltpu.make_async_copy(
                table_hbm.at[idx_v.at[j]], rows_v.at[b], gsems[b]).wait()

        def o_fire(j, b):
            pltpu.async_copy(rows_v.at[b], out_hbm.at[base + j], osems[b])

        def o_wait(j, b):
            pltpu.make_async_copy(
                rows_v.at[b], out_hbm.at[base + j], osems[b]).wait()

        for b in range(_LA):
            g_fire(b, b)

        def step(j, b, first, last):
            b2 = (b + _LA) % _NB
            g_wait(j, b)
            o_fire(j, b)
            if not first:
                o_wait(j - _LA, b2)
            if not last:
                g_fire(j + _LA, b2)

        for b in range(_NB):
            step(b, b, first=(b < _LA), last=False)

        def group(g, carry):
            j0 = g * _NB
            for b in range(_NB):
                step(j0 + b, b, first=False, last=False)
            return carry

        lax.fori_loop(1, m_per_w // _NB - 1, group, 0)

        j0 = m_per_w - _NB
        for b in range(_NB):
            step(j0 + b, b, first=False, last=(b >= _NB - _LA))

        for j in range(m_per_w - _LA, m_per_w):
            o_wait(j, j % _NB)

    return emb


def kernel(X, E):
    M, S = X.shape
    D = E.shape[1]
    return _emb_call(M, S, D)(E, X)


# speedup vs baseline: 2.0484x; 1.3645x over previous
import functools

import jax
import jax.numpy as jnp
from jax import lax
from jax.experimental import pallas as pl
from jax.experimental.pallas import tpu as pltpu
from jax.experimental.pallas import tpu_sc as plsc

_NW = 32
_NB = 8
_LA = 4


def _emb_call(M, S, D, DP):
    m_per_w = M // _NW
    assert m_per_w % _NB == 0 and m_per_w >= 2 * _NB
    mesh = plsc.VectorSubcoreMesh(core_axis_name="c", subcore_axis_name="s")

    @functools.partial(
        pl.kernel,
        mesh=mesh,
        out_type=jax.ShapeDtypeStruct((M, S, DP), jnp.float32),
        scratch_types=[
            pltpu.VMEM((m_per_w, S), jnp.int32),
            pltpu.VMEM((_NB, S, D), jnp.float32),
            [pltpu.SemaphoreType.DMA] * _NB,
            [pltpu.SemaphoreType.DMA] * _NB,
        ],
        compiler_params=pltpu.CompilerParams(use_tc_tiling_on_sc=False),
    )
    def emb(table_hbm, idx_hbm, out_hbm, idx_v, rows_v, gsems, osems):
        wid = lax.axis_index("s") * 2 + lax.axis_index("c")
        base = wid * m_per_w
        pltpu.sync_copy(idx_hbm.at[pl.ds(base, m_per_w)], idx_v)

        def g_fire(j, b):
            pltpu.async_copy(table_hbm.at[idx_v.at[j]], rows_v.at[b], gsems[b])

        def g_wait(j, b):
            pltpu.make_async_copy(
                table_hbm.at[idx_v.at[j]], rows_v.at[b], gsems[b]).wait()

        def o_fire(j, b):
            pltpu.async_copy(
                rows_v.at[b], out_hbm.at[base + j].at[:, pl.ds(0, D)],
                osems[b])

        def o_wait(j, b):
            pltpu.make_async_copy(
                rows_v.at[b], out_hbm.at[base + j].at[:, pl.ds(0, D)],
                osems[b]).wait()

        for b in range(_LA):
            g_fire(b, b)

        def step(j, b, first, last):
            b2 = (b + _LA) % _NB
            g_wait(j, b)
            o_fire(j, b)
            if not first:
                o_wait(j - _LA, b2)
            if not last:
                g_fire(j + _LA, b2)

        for b in range(_NB):
            step(b, b, first=(b < _LA), last=False)

        def group(g, carry):
            j0 = g * _NB
            for b in range(_NB):
                step(j0 + b, b, first=False, last=False)
            return carry

        lax.fori_loop(1, m_per_w // _NB - 1, group, 0)

        j0 = m_per_w - _NB
        for b in range(_NB):
            step(j0 + b, b, first=False, last=(b >= _NB - _LA))

        for j in range(m_per_w - _LA, m_per_w):
            o_wait(j, j % _NB)

    return emb


def kernel(X, E):
    M, S = X.shape
    V, D = E.shape
    out_pad = _emb_call(M, S, D, 128)(E, X)
    return lax.slice(out_pad, (0, 0, 0), (M, S, D))
